# baseline (speedup 1.0000x reference)
import functools
import math

import jax
import jax.numpy as jnp
from jax import lax
from jax.experimental import pallas as pl
from jax.experimental.pallas import tpu as pltpu
from jax.experimental.pallas import tpu_sc as plsc

NC = 2
NS = 16
NW = NC * NS
C_E = 128


def _qkv_body(nblk, inv_sqrt, x_ref, wq_ref, wk_ref, wv_ref, qb_ref, vb_ref,
              q_ref, k_ref, v_ref, mb_ref, smq, smk):
    i = pl.program_id(0)
    xb = x_ref[...]
    dn = (((1,), (1,)), ((), ()))
    q = lax.dot_general(xb, wq_ref[...], dn,
                        preferred_element_type=jnp.float32) + qb_ref[...]
    q_ref[...] = q.astype(jnp.bfloat16)
    col = lax.broadcasted_iota(jnp.int32, wk_ref.shape, 1)
    wkm = jnp.where(col == 0, 0.0, wk_ref[...])
    k = lax.dot_general(xb, wkm, dn, preferred_element_type=jnp.float32)
    v = lax.dot_general(xb, wv_ref[...], dn,
                        preferred_element_type=jnp.float32) + vb_ref[...]
    k_ref[...] = k.astype(jnp.bfloat16)
    v_ref[...] = v.astype(jnp.bfloat16)
    ones = jnp.ones((q.shape[1], 1), jnp.float32)
    cdn = (((1,), (0,)), ((), ()))
    mq = jnp.max(lax.dot_general(q * q, ones, cdn,
                                 preferred_element_type=jnp.float32))
    mk = jnp.max(lax.dot_general(k * k, ones, cdn,
                                 preferred_element_type=jnp.float32))
    smq[0, 0] = jnp.maximum(jnp.where(i == 0, 0.0, smq[0, 0]), mq)
    smk[0, 0] = jnp.maximum(jnp.where(i == 0, 0.0, smk[0, 0]), mk)

    @pl.when(i == nblk - 1)
    def _():
        bound = jnp.sqrt(smq[0, 0]) * jnp.sqrt(smk[0, 0]) * inv_sqrt
        mb_ref[...] = jnp.full((1, 128), bound, jnp.float32)


def _pool_body(nblk, num_graphs, hp_ref, b_ref, psum_ref, fcw_ref,
               fcb_ref, out_ref, seg_acc, cnt_acc):
    i = pl.program_id(0)

    @pl.when(i == 0)
    def _():
        seg_acc[...] = jnp.zeros_like(seg_acc)
        cnt_acc[...] = jnp.zeros_like(cnt_acc)

    hp = hp_ref[...].astype(jnp.float32)
    h = jnp.maximum(hp[0] + hp[1], 0.0)
    r_blk = h.shape[0]
    gi = lax.broadcasted_iota(jnp.int32, (r_blk, num_graphs), 1)
    onehot = jnp.where(b_ref[...] == gi, 1.0, 0.0)
    dn = (((0,), (0,)), ((), ()))
    seg_acc[...] += lax.dot_general(onehot, h, dn,
                                    preferred_element_type=jnp.float32)
    cnt_acc[...] += lax.dot_general(onehot, jnp.ones((r_blk, 1), jnp.float32),
                                    dn, preferred_element_type=jnp.float32)

    @pl.when(i == nblk - 1)
    def _():
        z = jnp.sum(psum_ref[...])
        pooled = seg_acc[...] / jnp.maximum(cnt_acc[...], 1.0) / z
        out_ref[...] = lax.dot_general(
            pooled, fcw_ref[...], (((1,), (1,)), ((), ())),
            preferred_element_type=jnp.float32) + fcb_ref[...]


def kernel(x, edge_index, batch, Wq_w, Wq_b, W_key, Wv_w, Wv_b, fc_w, fc_b):
    n, in_ch = x.shape
    hid = Wq_w.shape[0]
    num_classes = fc_w.shape[0]
    num_graphs = 64
    e_real = edge_index.shape[1] + n
    epw = ((e_real + NW - 1) // NW + 2 * C_E - 1) // (2 * C_E) * (2 * C_E)
    et_pad = epw * NW
    nch = epw // C_E
    ng = hid // 16
    inv_sqrt = 1.0 / math.sqrt(float(hid))

    e_raw = edge_index.shape[1]
    assert e_raw % C_E == 0
    zeros_h = jnp.zeros((n, hid), jnp.bfloat16)

    r_blk = 400
    nblk = n // r_blk
    full = pl.BlockSpec((in_ch, hid), lambda i: (0, 0))
    brow = pl.BlockSpec((1, hid), lambda i: (0, 0))
    q, k, v, mb = pl.pallas_call(
        functools.partial(_qkv_body, nblk, inv_sqrt),
        grid=(nblk,),
        in_specs=[pl.BlockSpec((r_blk, in_ch), lambda i: (i, 0)),
                  full, full, full, brow, brow],
        out_specs=[pl.BlockSpec((r_blk, hid), lambda i: (i, 0))] * 3 +
                  [pl.BlockSpec((1, 128), lambda i: (0, 0))],
        out_shape=[jax.ShapeDtypeStruct((n, hid), jnp.bfloat16)] * 3 +
                  [jax.ShapeDtypeStruct((1, 128), jnp.float32)],
        scratch_shapes=[pltpu.SMEM((1, 1), jnp.float32),
                        pltpu.SMEM((1, 1), jnp.float32)],
        compiler_params=pltpu.CompilerParams(
            dimension_semantics=("arbitrary",)),
    )(x, Wq_w, W_key, Wv_w, Wq_b.reshape(1, hid), Wv_b.reshape(1, hid))

    hw = hid // 2
    q_pk = lax.bitcast_convert_type(q.reshape(n, hw, 2), jnp.float32)
    k_pk = lax.bitcast_convert_type(k.reshape(n, hw, 2), jnp.float32)
    v_pk = lax.bitcast_convert_type(v.reshape(n, hw, 2), jnp.float32)

    mesh = plsc.VectorSubcoreMesh(core_axis_name="c", subcore_axis_name="s")

    @functools.partial(
        pl.kernel,
        out_type=[jax.ShapeDtypeStruct((NC, n, hid), jnp.bfloat16),
                  jax.ShapeDtypeStruct((NW * 128,), jnp.float32)],
        mesh=mesh,
        compiler_params=pltpu.CompilerParams(needs_layout_passes=False,
                                             use_tc_tiling_on_sc=False),
        scratch_types=[
            [pltpu.VMEM((C_E,), jnp.int32)] * 2,
            [pltpu.VMEM((C_E,), jnp.int32)] * 2,
            [pltpu.VMEM((C_E, hid // 2), jnp.float32)] * 2,
            [pltpu.VMEM((C_E, hid // 2), jnp.float32)] * 2,
            [pltpu.VMEM((C_E, hid // 2), jnp.float32)] * 2,
            pltpu.VMEM((C_E * 16,), jnp.float32),
            pltpu.VMEM((C_E,), jnp.float32),
            [pltpu.VMEM((C_E, hid), jnp.bfloat16)] * 2,
            [pltpu.VMEM((C_E,), jnp.int32)] * 2,
            pltpu.VMEM((128,), jnp.float32),
            pltpu.VMEM((128,), jnp.float32),
            pltpu.VMEM_SHARED((n, hid), jnp.bfloat16),
            [pltpu.SemaphoreType.DMA] * 2,
            [pltpu.SemaphoreType.DMA] * 2,
            [pltpu.SemaphoreType.DMA] * 2,
            [pltpu.SemaphoreType.DMA] * 2,
            [pltpu.SemaphoreType.DMA] * 2,
        ])
    def _edge_stage(q_hbm, k_hbm, v_hbm, ei_hbm, mb_hbm, zeros_hbm,
                    hpart_hbm, psum_hbm,
                    idx_d, idx_s, qrows, krows, vrows, partials, pbuf, msg,
                    sidx, mbuf, sumbuf, hsh, semq, semk, semv, ssem, isem):
        cid = lax.axis_index("c")
        sid = lax.axis_index("s")
        wid = sid * NC + cid
        base = pl.multiple_of(wid * epw, 128)
        rpt = (n // NS + 15) // 16 * 16
        tail = n - (NS - 1) * rpt
        r0 = pl.multiple_of(sid * rpt, 16)

        @pl.when(sid < NS - 1)
        def _():
            pltpu.sync_copy(zeros_hbm.at[pl.ds(r0, rpt)],
                            hsh.at[pl.ds(r0, rpt)])

        @pl.when(sid == NS - 1)
        def _():
            pltpu.sync_copy(zeros_hbm.at[pl.ds((NS - 1) * rpt, tail)],
                            hsh.at[pl.ds((NS - 1) * rpt, tail)])

        pltpu.sync_copy(mb_hbm, mbuf)
        m = jnp.max(mbuf[pl.ds(0, 16)]) - 40.0
        plsc.subcore_barrier()

        def issue(ci, b):
            off = pl.multiple_of(base + ci * C_E, 128)

            @pl.when(off < e_raw)
            def _():
                cp1 = pltpu.async_copy(ei_hbm.at[1, pl.ds(off, C_E)],
                                       idx_d[b], isem[b])
                cp2 = pltpu.async_copy(ei_hbm.at[0, pl.ds(off, C_E)],
                                       idx_s[b], isem[b])
                cp1.wait()
                cp2.wait()

            @pl.when(off >= e_raw)
            def _():
                for i in range(C_E // 16):
                    eids = off + i * 16 + lax.iota(jnp.int32, 16)
                    val = jnp.minimum(eids - e_raw, n - 1)
                    idx_d[b][pl.ds(i * 16, 16)] = val
                    idx_s[b][pl.ds(i * 16, 16)] = val

            pltpu.async_copy(q_hbm.at[idx_d[b]], qrows[b], semq[b])
            pltpu.async_copy(k_hbm.at[idx_s[b]], krows[b], semk[b])
            pltpu.async_copy(v_hbm.at[idx_s[b]], vrows[b], semv[b])

        def compute(ci, b, acc, drain_scatter):
            off = pl.multiple_of(base + ci * C_E, 128)
            pltpu.make_async_copy(q_hbm.at[idx_d[b]], qrows[b], semq[b]).wait()
            pltpu.make_async_copy(k_hbm.at[idx_s[b]], krows[b], semk[b]).wait()

            @pl.loop(0, C_E, unroll=4)
            def _edge(ei):
                dacc = None
                for w in range(hid // 32):
                    qa, qb2 = plsc.unpack(
                        plsc.bitcast(qrows[b][ei, pl.ds(w * 16, 16)],
                                     jnp.bfloat16),
                        format=plsc.PackFormat.INTERLEAVED)
                    ka, kb2 = plsc.unpack(
                        plsc.bitcast(krows[b][ei, pl.ds(w * 16, 16)],
                                     jnp.bfloat16),
                        format=plsc.PackFormat.INTERLEAVED)
                    t = qa * ka + qb2 * kb2
                    dacc = t if dacc is None else dacc + t
                partials[pl.ds(ei * 16, 16)] = dacc

            @pl.loop(0, C_E // 16, init_carry=acc)
            def acc(grp, a):
                flat = (grp * 16 + lax.iota(jnp.int32, 16)) * 16
                acc2 = plsc.load_gather(partials, [flat])
                for j in range(1, 16):
                    acc2 = acc2 + plsc.load_gather(partials, [flat + j])
                pv = jnp.exp(acc2 * inv_sqrt - m)
                eids = off + grp * 16 + lax.iota(jnp.int32, 16)
                pv = jnp.where(eids < e_real, pv, 0.0)
                pbuf[pl.ds(grp * 16, 16)] = pv
                return a + pv

            pltpu.make_async_copy(v_hbm.at[idx_s[b]], vrows[b], semv[b]).wait()
            if drain_scatter:
                pltpu.make_async_copy(msg[b], hsh.at[sidx[b]], ssem[b]).wait()

            @pl.loop(0, C_E, unroll=4)
            def _scale(ei):
                pb = plsc.load_gather(pbuf, [jnp.full((16,), ei, jnp.int32)])
                pb_bf = plsc.pack(pb, pb, format=plsc.PackFormat.INTERLEAVED)
                for w in range(hid // 32):
                    vw = plsc.bitcast(vrows[b][ei, pl.ds(w * 16, 16)],
                                      jnp.bfloat16)
                    msg[b][ei, pl.ds(w * 32, 32)] = vw * pb_bf

            for i in range(C_E // 16):
                sidx[b][pl.ds(i * 16, 16)] = idx_d[b][pl.ds(i * 16, 16)]
            pltpu.async_copy(msg[b], hsh.at[sidx[b]], ssem[b], add=True)
            return acc

        issue(0, 0)
        issue(1, 1)
        acc0 = compute(0, 0, jnp.zeros((16,), jnp.float32), False)
        issue(2, 0)
        acc0 = compute(1, 1, acc0, False)

        @pl.loop(1, nch // 2 - 1, init_carry=acc0)
        def psum_vec(g, acc):
            for b in (0, 1):
                ci = g * 2 + b
                issue(ci + 1, 1 - b)
                acc = compute(ci, b, acc, True)
            return acc

        issue(nch - 1, 1)
        psum_vec = compute(nch - 2, 0, psum_vec, True)
        psum_vec = compute(nch - 1, 1, psum_vec, True)
        pltpu.make_async_copy(msg[0], hsh.at[sidx[0]], ssem[0]).wait()
        pltpu.make_async_copy(msg[1], hsh.at[sidx[1]], ssem[1]).wait()

        sumbuf[pl.ds(0, 16)] = psum_vec
        for j in range(1, 8):
            sumbuf[pl.ds(j * 16, 16)] = jnp.zeros((16,), jnp.float32)
        pltpu.sync_copy(sumbuf, psum_hbm.at[pl.ds(pl.multiple_of(wid * 128,
                                                                 128), 128)])
        plsc.subcore_barrier()

        @pl.when(sid < NS - 1)
        def _():
            pltpu.sync_copy(hsh.at[pl.ds(r0, rpt)],
                            hpart_hbm.at[cid, pl.ds(r0, rpt)])

        @pl.when(sid == NS - 1)
        def _():
            pltpu.sync_copy(hsh.at[pl.ds((NS - 1) * rpt, tail)],
                            hpart_hbm.at[cid, pl.ds((NS - 1) * rpt, tail)])

    hpart, psum = _edge_stage(q_pk, k_pk, v_pk, edge_index, mb.reshape(128),
                              zeros_h)

    logits = pl.pallas_call(
        functools.partial(_pool_body, nblk, num_graphs),
        grid=(nblk,),
        in_specs=[pl.BlockSpec((NC, r_blk, hid), lambda i: (0, i, 0)),
                  pl.BlockSpec((r_blk, 1), lambda i: (i, 0)),
                  pl.BlockSpec((NW, 128), lambda i: (0, 0)),
                  pl.BlockSpec((num_classes, hid), lambda i: (0, 0)),
                  pl.BlockSpec((1, num_classes), lambda i: (0, 0))],
        out_specs=pl.BlockSpec((num_graphs, num_classes), lambda i: (0, 0)),
        out_shape=jax.ShapeDtypeStruct((num_graphs, num_classes), jnp.float32),
        scratch_shapes=[pltpu.VMEM((num_graphs, hid), jnp.float32),
                        pltpu.VMEM((num_graphs, 1), jnp.float32)],
        compiler_params=pltpu.CompilerParams(
            dimension_semantics=("arbitrary",)),
    )(hpart, batch.reshape(n, 1), psum.reshape(NW, 128), fc_w,
      fc_b.reshape(1, num_classes))
    return logits


# speedup vs baseline: 1.0561x; 1.0561x over previous
import functools
import math

import jax
import jax.numpy as jnp
from jax import lax
from jax.experimental import pallas as pl
from jax.experimental.pallas import tpu as pltpu
from jax.experimental.pallas import tpu_sc as plsc

NC = 2
NS = 16
NW = NC * NS
C_E = 128


def _qkv_body(nblk, inv_sqrt, x_ref, wq_ref, wk_ref, wv_ref, qb_ref, vb_ref,
              q_ref, k_ref, v_ref, mb_ref, smq, smk):
    i = pl.program_id(0)
    xb = x_ref[...]
    dn = (((1,), (1,)), ((), ()))
    q = lax.dot_general(xb, wq_ref[...], dn,
                        preferred_element_type=jnp.float32) + qb_ref[...]
    q_ref[...] = q.astype(jnp.bfloat16)
    col = lax.broadcasted_iota(jnp.int32, wk_ref.shape, 1)
    wkm = jnp.where(col == 0, 0.0, wk_ref[...])
    k = lax.dot_general(xb, wkm, dn, preferred_element_type=jnp.float32)
    v = lax.dot_general(xb, wv_ref[...], dn,
                        preferred_element_type=jnp.float32) + vb_ref[...]
    k_ref[...] = k.astype(jnp.bfloat16)
    v_ref[...] = v.astype(jnp.bfloat16)
    ones = jnp.ones((q.shape[1], 1), jnp.float32)
    cdn = (((1,), (0,)), ((), ()))
    mq = jnp.max(lax.dot_general(q * q, ones, cdn,
                                 preferred_element_type=jnp.float32))
    mk = jnp.max(lax.dot_general(k * k, ones, cdn,
                                 preferred_element_type=jnp.float32))
    smq[0, 0] = jnp.maximum(jnp.where(i == 0, 0.0, smq[0, 0]), mq)
    smk[0, 0] = jnp.maximum(jnp.where(i == 0, 0.0, smk[0, 0]), mk)

    @pl.when(i == nblk - 1)
    def _():
        bound = jnp.sqrt(smq[0, 0]) * jnp.sqrt(smk[0, 0]) * inv_sqrt
        mb_ref[...] = jnp.full((1, 128), bound, jnp.float32)


def _pool_body(nblk, num_graphs, h0_ref, h1_ref, b_ref, psum_ref, fcw_ref,
               fcb_ref, out_ref, seg_acc, cnt_acc):
    i = pl.program_id(0)

    @pl.when(i == 0)
    def _():
        seg_acc[...] = jnp.zeros_like(seg_acc)
        cnt_acc[...] = jnp.zeros_like(cnt_acc)

    h = jnp.maximum(h0_ref[...].astype(jnp.float32) +
                    h1_ref[...].astype(jnp.float32), 0.0)
    r_blk = h.shape[0]
    gi = lax.broadcasted_iota(jnp.int32, (r_blk, num_graphs), 1)
    onehot = jnp.where(b_ref[...] == gi, 1.0, 0.0)
    dn = (((0,), (0,)), ((), ()))
    seg_acc[...] += lax.dot_general(onehot, h, dn,
                                    preferred_element_type=jnp.float32)
    cnt_acc[...] += lax.dot_general(onehot, jnp.ones((r_blk, 1), jnp.float32),
                                    dn, preferred_element_type=jnp.float32)

    @pl.when(i == nblk - 1)
    def _():
        z = jnp.sum(psum_ref[...])
        pooled = seg_acc[...] / jnp.maximum(cnt_acc[...], 1.0) / z
        out_ref[...] = lax.dot_general(
            pooled, fcw_ref[...], (((1,), (1,)), ((), ())),
            preferred_element_type=jnp.float32) + fcb_ref[...]


def kernel(x, edge_index, batch, Wq_w, Wq_b, W_key, Wv_w, Wv_b, fc_w, fc_b):
    n, in_ch = x.shape
    hid = Wq_w.shape[0]
    num_classes = fc_w.shape[0]
    num_graphs = 64
    e_real = edge_index.shape[1] + n
    epw = ((e_real + NW - 1) // NW + 2 * C_E - 1) // (2 * C_E) * (2 * C_E)
    et_pad = epw * NW
    nch = epw // C_E
    ng = hid // 16
    inv_sqrt = 1.0 / math.sqrt(float(hid))

    e_raw = edge_index.shape[1]
    assert e_raw % C_E == 0
    zeros_h = jnp.zeros((n, hid), jnp.bfloat16)

    r_blk = 400
    nblk = n // r_blk
    full = pl.BlockSpec((in_ch, hid), lambda i: (0, 0))
    brow = pl.BlockSpec((1, hid), lambda i: (0, 0))
    q, k, v, mb = pl.pallas_call(
        functools.partial(_qkv_body, nblk, inv_sqrt),
        grid=(nblk,),
        in_specs=[pl.BlockSpec((r_blk, in_ch), lambda i: (i, 0)),
                  full, full, full, brow, brow],
        out_specs=[pl.BlockSpec((r_blk, hid), lambda i: (i, 0))] * 3 +
                  [pl.BlockSpec((1, 128), lambda i: (0, 0))],
        out_shape=[jax.ShapeDtypeStruct((n, hid), jnp.bfloat16)] * 3 +
                  [jax.ShapeDtypeStruct((1, 128), jnp.float32)],
        scratch_shapes=[pltpu.SMEM((1, 1), jnp.float32),
                        pltpu.SMEM((1, 1), jnp.float32)],
        compiler_params=pltpu.CompilerParams(
            dimension_semantics=("arbitrary",)),
    )(x, Wq_w, W_key, Wv_w, Wq_b.reshape(1, hid), Wv_b.reshape(1, hid))

    hw = hid // 2
    q_pk = lax.bitcast_convert_type(q.reshape(n, hw, 2), jnp.float32)
    k_pk = lax.bitcast_convert_type(k.reshape(n, hw, 2), jnp.float32)
    v_pk = lax.bitcast_convert_type(v.reshape(n, hw, 2), jnp.float32)

    mesh = plsc.VectorSubcoreMesh(core_axis_name="c", subcore_axis_name="s")

    @functools.partial(
        pl.kernel,
        out_type=[jax.ShapeDtypeStruct((NC, n, hid), jnp.bfloat16),
                  jax.ShapeDtypeStruct((NW * 128,), jnp.float32)],
        mesh=mesh,
        compiler_params=pltpu.CompilerParams(needs_layout_passes=False,
                                             use_tc_tiling_on_sc=False),
        scratch_types=[
            [pltpu.VMEM((C_E,), jnp.int32)] * 2,
            [pltpu.VMEM((C_E,), jnp.int32)] * 2,
            [pltpu.VMEM((C_E, hid // 2), jnp.float32)] * 2,
            [pltpu.VMEM((C_E, hid // 2), jnp.float32)] * 2,
            [pltpu.VMEM((C_E, hid // 2), jnp.float32)] * 2,
            pltpu.VMEM((C_E * 16,), jnp.float32),
            pltpu.VMEM((C_E,), jnp.float32),
            [pltpu.VMEM((C_E, hid), jnp.bfloat16)] * 2,
            [pltpu.VMEM((C_E,), jnp.int32)] * 2,
            pltpu.VMEM((128,), jnp.float32),
            pltpu.VMEM((128,), jnp.float32),
            pltpu.VMEM_SHARED((n, hid), jnp.bfloat16),
            [pltpu.SemaphoreType.DMA] * 2,
            [pltpu.SemaphoreType.DMA] * 2,
            [pltpu.SemaphoreType.DMA] * 2,
            [pltpu.SemaphoreType.DMA] * 2,
            [pltpu.SemaphoreType.DMA] * 2,
        ])
    def _edge_stage(q_hbm, k_hbm, v_hbm, ei_hbm, mb_hbm, zeros_hbm,
                    hpart_hbm, psum_hbm,
                    idx_d, idx_s, qrows, krows, vrows, partials, pbuf, msg,
                    sidx, mbuf, sumbuf, hsh, semq, semk, semv, ssem, isem):
        cid = lax.axis_index("c")
        sid = lax.axis_index("s")
        wid = sid * NC + cid
        base = pl.multiple_of(wid * epw, 128)
        rpt = (n // NS + 15) // 16 * 16
        tail = n - (NS - 1) * rpt
        r0 = pl.multiple_of(sid * rpt, 16)

        @pl.when(sid < NS - 1)
        def _():
            pltpu.sync_copy(zeros_hbm.at[pl.ds(r0, rpt)],
                            hsh.at[pl.ds(r0, rpt)])

        @pl.when(sid == NS - 1)
        def _():
            pltpu.sync_copy(zeros_hbm.at[pl.ds((NS - 1) * rpt, tail)],
                            hsh.at[pl.ds((NS - 1) * rpt, tail)])

        pltpu.sync_copy(mb_hbm, mbuf)
        m = jnp.max(mbuf[pl.ds(0, 16)]) - 40.0
        plsc.subcore_barrier()

        def issue(ci, b):
            off = pl.multiple_of(base + ci * C_E, 128)

            @pl.when(off < e_raw)
            def _():
                cp1 = pltpu.async_copy(ei_hbm.at[1, pl.ds(off, C_E)],
                                       idx_d[b], isem[b])
                cp2 = pltpu.async_copy(ei_hbm.at[0, pl.ds(off, C_E)],
                                       idx_s[b], isem[b])
                cp1.wait()
                cp2.wait()

            @pl.when(off >= e_raw)
            def _():
                for i in range(C_E // 16):
                    eids = off + i * 16 + lax.iota(jnp.int32, 16)
                    val = jnp.minimum(eids - e_raw, n - 1)
                    idx_d[b][pl.ds(i * 16, 16)] = val
                    idx_s[b][pl.ds(i * 16, 16)] = val

            pltpu.async_copy(q_hbm.at[idx_d[b]], qrows[b], semq[b])
            pltpu.async_copy(k_hbm.at[idx_s[b]], krows[b], semk[b])
            pltpu.async_copy(v_hbm.at[idx_s[b]], vrows[b], semv[b])

        def compute(ci, b, acc, drain_scatter):
            off = pl.multiple_of(base + ci * C_E, 128)
            pltpu.make_async_copy(q_hbm.at[idx_d[b]], qrows[b], semq[b]).wait()
            pltpu.make_async_copy(k_hbm.at[idx_s[b]], krows[b], semk[b]).wait()

            @pl.loop(0, C_E, unroll=4)
            def _edge(ei):
                dacc = None
                for w in range(hid // 32):
                    qa, qb2 = plsc.unpack(
                        plsc.bitcast(qrows[b][ei, pl.ds(w * 16, 16)],
                                     jnp.bfloat16),
                        format=plsc.PackFormat.INTERLEAVED)
                    ka, kb2 = plsc.unpack(
                        plsc.bitcast(krows[b][ei, pl.ds(w * 16, 16)],
                                     jnp.bfloat16),
                        format=plsc.PackFormat.INTERLEAVED)
                    t = qa * ka + qb2 * kb2
                    dacc = t if dacc is None else dacc + t
                partials[pl.ds(ei * 16, 16)] = dacc

            @pl.loop(0, C_E // 16, init_carry=acc)
            def acc(grp, a):
                flat = (grp * 16 + lax.iota(jnp.int32, 16)) * 16
                acc2 = plsc.load_gather(partials, [flat])
                for j in range(1, 16):
                    acc2 = acc2 + plsc.load_gather(partials, [flat + j])
                pv = jnp.exp(acc2 * inv_sqrt - m)
                eids = off + grp * 16 + lax.iota(jnp.int32, 16)
                pv = jnp.where(eids < e_real, pv, 0.0)
                pbuf[pl.ds(grp * 16, 16)] = pv
                return a + pv

            pltpu.make_async_copy(v_hbm.at[idx_s[b]], vrows[b], semv[b]).wait()
            if drain_scatter:
                pltpu.make_async_copy(msg[b], hsh.at[sidx[b]], ssem[b]).wait()

            @pl.loop(0, C_E, unroll=4)
            def _scale(ei):
                pb = plsc.load_gather(pbuf, [jnp.full((16,), ei, jnp.int32)])
                pb_bf = plsc.pack(pb, pb, format=plsc.PackFormat.INTERLEAVED)
                for w in range(hid // 32):
                    vw = plsc.bitcast(vrows[b][ei, pl.ds(w * 16, 16)],
                                      jnp.bfloat16)
                    msg[b][ei, pl.ds(w * 32, 32)] = vw * pb_bf

            for i in range(C_E // 16):
                sidx[b][pl.ds(i * 16, 16)] = idx_d[b][pl.ds(i * 16, 16)]
            pltpu.async_copy(msg[b], hsh.at[sidx[b]], ssem[b], add=True)
            return acc

        issue(0, 0)
        issue(1, 1)
        acc0 = compute(0, 0, jnp.zeros((16,), jnp.float32), False)
        issue(2, 0)
        acc0 = compute(1, 1, acc0, False)

        @pl.loop(1, nch // 2 - 1, init_carry=acc0)
        def psum_vec(g, acc):
            for b in (0, 1):
                ci = g * 2 + b
                issue(ci + 1, 1 - b)
                acc = compute(ci, b, acc, True)
            return acc

        issue(nch - 1, 1)
        psum_vec = compute(nch - 2, 0, psum_vec, True)
        psum_vec = compute(nch - 1, 1, psum_vec, True)
        pltpu.make_async_copy(msg[0], hsh.at[sidx[0]], ssem[0]).wait()
        pltpu.make_async_copy(msg[1], hsh.at[sidx[1]], ssem[1]).wait()

        sumbuf[pl.ds(0, 16)] = psum_vec
        for j in range(1, 8):
            sumbuf[pl.ds(j * 16, 16)] = jnp.zeros((16,), jnp.float32)
        pltpu.sync_copy(sumbuf, psum_hbm.at[pl.ds(pl.multiple_of(wid * 128,
                                                                 128), 128)])
        plsc.subcore_barrier()

        @pl.when(sid < NS - 1)
        def _():
            pltpu.sync_copy(hsh.at[pl.ds(r0, rpt)],
                            hpart_hbm.at[cid, pl.ds(r0, rpt)])

        @pl.when(sid == NS - 1)
        def _():
            pltpu.sync_copy(hsh.at[pl.ds((NS - 1) * rpt, tail)],
                            hpart_hbm.at[cid, pl.ds((NS - 1) * rpt, tail)])

    hpart, psum = _edge_stage(q_pk, k_pk, v_pk, edge_index, mb.reshape(128),
                              zeros_h)

    logits = pl.pallas_call(
        functools.partial(_pool_body, nblk, num_graphs),
        grid=(nblk,),
        in_specs=[pl.BlockSpec((r_blk, hid), lambda i: (i, 0)),
                  pl.BlockSpec((r_blk, hid), lambda i: (i, 0)),
                  pl.BlockSpec((r_blk, 1), lambda i: (i, 0)),
                  pl.BlockSpec((NW, 128), lambda i: (0, 0)),
                  pl.BlockSpec((num_classes, hid), lambda i: (0, 0)),
                  pl.BlockSpec((1, num_classes), lambda i: (0, 0))],
        out_specs=pl.BlockSpec((num_graphs, num_classes), lambda i: (0, 0)),
        out_shape=jax.ShapeDtypeStruct((num_graphs, num_classes), jnp.float32),
        scratch_shapes=[pltpu.VMEM((num_graphs, hid), jnp.float32),
                        pltpu.VMEM((num_graphs, 1), jnp.float32)],
        compiler_params=pltpu.CompilerParams(
            dimension_semantics=("arbitrary",)),
    )(hpart[0], hpart[1], batch.reshape(n, 1), psum.reshape(NW, 128), fc_w,
      fc_b.reshape(1, num_classes))
    return logits
